# baseline (speedup 1.0000x reference)
import functools
import math

import jax
import jax.numpy as jnp
from jax import lax
from jax.experimental import pallas as pl
from jax.experimental.pallas import tpu as pltpu, tpu_sc as plsc

B = 16384
E = 384
EP = 256
EH = E - EP
H = 4
DH = E // H
K = 1024
S = 3

NC = 2
NS = 16
NW = NC * NS


def _sc_gather(t0, t1, t2, cb0, cb1, cb2, rows):
    B_PER_W = rows // NW
    CH = min(128, B_PER_W)
    NCHUNK = B_PER_W // CH
    mesh = plsc.VectorSubcoreMesh(
        core_axis_name="c", subcore_axis_name="s",
        num_cores=NC, num_subcores=NS)

    @functools.partial(
        pl.kernel,
        out_type=jax.ShapeDtypeStruct((S, rows, EP), jnp.int32),
        mesh=mesh,
        scratch_types=[
            pltpu.VMEM((B_PER_W,), jnp.int32),
            pltpu.VMEM((B_PER_W,), jnp.int32),
            pltpu.VMEM((B_PER_W,), jnp.int32),
            pltpu.VMEM((CH, EP), jnp.int32),
            pltpu.VMEM((CH, EP), jnp.int32),
            pltpu.SemaphoreType.DMA,
            pltpu.SemaphoreType.DMA,
            pltpu.SemaphoreType.DMA,
            pltpu.SemaphoreType.DMA,
        ],
    )
    def gather_kernel(t0_hbm, t1_hbm, t2_hbm, cb0_hbm, cb1_hbm, cb2_hbm,
                      out_hbm, idx0_v, idx1_v, idx2_v, buf0, buf1,
                      sg0, sg1, sw0, sw1):
        wid = lax.axis_index("s") * NC + lax.axis_index("c")
        base = wid * B_PER_W
        tok_refs = (t0_hbm, t1_hbm, t2_hbm)
        cb_refs = (cb0_hbm, cb1_hbm, cb2_hbm)
        idx_refs = (idx0_v, idx1_v, idx2_v)
        for ti in range(S):
            pltpu.sync_copy(tok_refs[ti].at[pl.ds(base, B_PER_W)],
                            idx_refs[ti])

        bufs = (buf0, buf1)
        gsems = (sg0, sg1)
        wsems = (sw0, sw1)
        chunks = [(ti, c) for ti in range(S) for c in range(NCHUNK)]
        n_total = len(chunks)

        def start_gather(n):
            ti, c = chunks[n]
            return pltpu.async_copy(
                cb_refs[ti].at[idx_refs[ti].at[pl.ds(c * CH, CH)]],
                bufs[n % 2], gsems[n % 2])

        def start_write(n):
            ti, c = chunks[n]
            return pltpu.async_copy(
                bufs[n % 2],
                out_hbm.at[ti, pl.ds(base + c * CH, CH)],
                wsems[n % 2])

        cp_g = start_gather(0)
        cp_w = [None, None]
        for n in range(n_total):
            nxt = None
            if n + 1 < n_total:
                nb = (n + 1) % 2
                if cp_w[nb] is not None:
                    cp_w[nb].wait()
                    cp_w[nb] = None
                nxt = start_gather(n + 1)
            cp_g.wait()
            cp_w[n % 2] = start_write(n)
            cp_g = nxt
        for w in cp_w:
            if w is not None:
                w.wait()

    return gather_kernel(t0, t1, t2, cb0, cb1, cb2)


BB = 2048
EPS = 1e-5
SCALE = 1.0 / math.sqrt(DH)


def _tc_body(g_ref, wlo_ref, whi_ref, bqkv_ref, hm_ref, hmt_ref, wo_ref,
             bo_ref, gamma_ref, beta_ref, prev_ref, out_ref):
    gi = g_ref[...].reshape(S * BB, EP)
    elo = lax.bitcast_convert_type(gi << 16, jnp.float32)
    ehi = lax.bitcast_convert_type(gi & jnp.int32(-65536),
                                   jnp.float32)[:, :EH]
    e = jnp.concatenate([elo, ehi], axis=1)
    qkv = (jnp.dot(elo.astype(jnp.bfloat16), wlo_ref[...],
                   preferred_element_type=jnp.float32)
           + jnp.dot(ehi.astype(jnp.bfloat16), whi_ref[...],
                     preferred_element_type=jnp.float32)
           ).astype(jnp.bfloat16) + bqkv_ref[...]
    q = [qkv[i * BB:(i + 1) * BB, 0:E] for i in range(S)]
    k = [qkv[i * BB:(i + 1) * BB, E:2 * E] for i in range(S)]
    v = [qkv[i * BB:(i + 1) * BB, 2 * E:3 * E] for i in range(S)]

    hm = hm_ref[...]
    hmt = hmt_ref[...]
    s = [[jnp.dot(q[i] * k[j], hm,
                  preferred_element_type=jnp.float32) * SCALE
          for j in range(S)] for i in range(S)]

    out_pre = []
    for i in range(S):
        ex = [jnp.exp(s[i][j]) for j in range(S)]
        den = ex[0] + ex[1] + ex[2]
        acc = jnp.zeros((BB, E), jnp.float32)
        for j in range(S):
            a = (ex[j] / den).astype(jnp.bfloat16)
            aexp = jnp.dot(a, hmt,
                           preferred_element_type=jnp.float32
                           ).astype(jnp.bfloat16)
            acc = acc + (aexp * v[j]).astype(jnp.float32)
        out_pre.append(acc)

    op = jnp.concatenate(out_pre, axis=0)
    x = jnp.dot(op.astype(jnp.bfloat16), wo_ref[...],
                preferred_element_type=jnp.float32) + bo_ref[...] + e
    mu = jnp.mean(x, axis=-1, keepdims=True)
    xc = x - mu
    var = jnp.mean(xc * xc, axis=-1, keepdims=True)
    y = xc * lax.rsqrt(var + EPS) * gamma_ref[...] + beta_ref[...]
    out_ref[...] = (y[0:BB] + y[BB:2 * BB] + y[2 * BB:3 * BB]) * (1.0 / 3.0)


def _tc_compute(g, wlo16, whi16, bqkv, hm16, hmt16, wo16, bo2, gamma2, beta2,
                prev, block0):
    rows = g.shape[1]
    n_blocks = rows // BB
    const = lambda b: (0, 0)
    return pl.pallas_call(
        _tc_body,
        grid=(n_blocks,),
        in_specs=[
            pl.BlockSpec((S, BB, EP), lambda b: (0, b, 0)),
            pl.BlockSpec((EP, 3 * E), const),
            pl.BlockSpec((EH, 3 * E), const),
            pl.BlockSpec((1, 3 * E), const),
            pl.BlockSpec((E, H), const),
            pl.BlockSpec((H, E), const),
            pl.BlockSpec((E, E), const),
            pl.BlockSpec((1, E), const),
            pl.BlockSpec((1, E), const),
            pl.BlockSpec((1, E), const),
            pl.BlockSpec(memory_space=pltpu.MemorySpace.HBM),
        ],
        out_specs=pl.BlockSpec((BB, E), lambda b, _b0=block0: (_b0 + b, 0)),
        out_shape=jax.ShapeDtypeStruct((B, E), jnp.float32),
        input_output_aliases={10: 0},
    )(g, wlo16, whi16, bqkv, hm16, hmt16, wo16, bo2, gamma2, beta2, prev)


NSPLIT = 4
CHUNK = B // NSPLIT


def _pack(cb):
    cb16 = cb.astype(jnp.bfloat16)
    lo = cb16[:, :EP]
    hi = jnp.pad(cb16[:, EP:], ((0, 0), (0, EP - EH)))
    return lax.bitcast_convert_type(jnp.stack([lo, hi], axis=-1), jnp.int32)


def kernel(tokens, cb0, cb1, cb2, Wq, bq, Wk, bk, Wv, bv, Wo, bo, gamma, beta):
    t0 = tokens[:, 0]
    t1 = tokens[:, 1]
    t2 = tokens[:, 2]
    cb0p = _pack(cb0)
    cb1p = _pack(cb1)
    cb2p = _pack(cb2)

    wqkv16 = jnp.concatenate([Wq.T, Wk.T, Wv.T], axis=1).astype(jnp.bfloat16)
    wlo16 = wqkv16[:EP]
    whi16 = wqkv16[EP:]
    bqkv = jnp.concatenate([bq, bk, bv]).reshape(1, 3 * E).astype(jnp.bfloat16)
    head_of = jnp.arange(E, dtype=jnp.int32) // DH
    hm = (head_of[:, None] == jnp.arange(H, dtype=jnp.int32)[None, :])
    hm16 = hm.astype(jnp.bfloat16)
    hmt16 = hm.T.astype(jnp.bfloat16)
    wo16 = Wo.T.astype(jnp.bfloat16)
    bo2 = bo.reshape(1, E)
    gamma2 = gamma.reshape(1, E)
    beta2 = beta.reshape(1, E)

    feat = jnp.zeros((B, E), jnp.float32)
    for c in range(NSPLIT):
        sl = slice(c * CHUNK, (c + 1) * CHUNK)
        g = _sc_gather(t0[sl], t1[sl], t2[sl], cb0p, cb1p, cb2p, CHUNK)
        feat = _tc_compute(g, wlo16, whi16, bqkv, hm16, hmt16, wo16,
                           bo2, gamma2, beta2, feat, c * (CHUNK // BB))
    return feat


# speedup vs baseline: 1.2361x; 1.2361x over previous
import functools
import math

import jax
import jax.numpy as jnp
from jax import lax
from jax.experimental import pallas as pl
from jax.experimental.pallas import tpu as pltpu, tpu_sc as plsc

B = 16384
E = 384
EP = 256
EH = E - EP
H = 4
DH = E // H
K = 1024
S = 3

NC = 2
NS = 16
NW = NC * NS


def _sc_gather(t0, t1, t2, cb0, cb1, cb2, rows):
    B_PER_W = rows // NW
    CH = min(128, B_PER_W)
    NCHUNK = B_PER_W // CH
    mesh = plsc.VectorSubcoreMesh(
        core_axis_name="c", subcore_axis_name="s",
        num_cores=NC, num_subcores=NS)

    @functools.partial(
        pl.kernel,
        out_type=jax.ShapeDtypeStruct((S, rows, EP), jnp.int32),
        mesh=mesh,
        scratch_types=[
            pltpu.VMEM((B_PER_W,), jnp.int32),
            pltpu.VMEM((B_PER_W,), jnp.int32),
            pltpu.VMEM((B_PER_W,), jnp.int32),
            pltpu.VMEM((CH, EP), jnp.int32),
            pltpu.VMEM((CH, EP), jnp.int32),
            pltpu.SemaphoreType.DMA,
            pltpu.SemaphoreType.DMA,
            pltpu.SemaphoreType.DMA,
            pltpu.SemaphoreType.DMA,
        ],
    )
    def gather_kernel(t0_hbm, t1_hbm, t2_hbm, cb0_hbm, cb1_hbm, cb2_hbm,
                      out_hbm, idx0_v, idx1_v, idx2_v, buf0, buf1,
                      sg0, sg1, sw0, sw1):
        wid = lax.axis_index("s") * NC + lax.axis_index("c")
        base = wid * B_PER_W
        tok_refs = (t0_hbm, t1_hbm, t2_hbm)
        cb_refs = (cb0_hbm, cb1_hbm, cb2_hbm)
        idx_refs = (idx0_v, idx1_v, idx2_v)
        for ti in range(S):
            pltpu.sync_copy(tok_refs[ti].at[pl.ds(base, B_PER_W)],
                            idx_refs[ti])

        bufs = (buf0, buf1)
        gsems = (sg0, sg1)
        wsems = (sw0, sw1)
        chunks = [(ti, c) for ti in range(S) for c in range(NCHUNK)]
        n_total = len(chunks)

        def start_gather(n):
            ti, c = chunks[n]
            return pltpu.async_copy(
                cb_refs[ti].at[idx_refs[ti].at[pl.ds(c * CH, CH)]],
                bufs[n % 2], gsems[n % 2])

        def start_write(n):
            ti, c = chunks[n]
            return pltpu.async_copy(
                bufs[n % 2],
                out_hbm.at[ti, pl.ds(base + c * CH, CH)],
                wsems[n % 2])

        cp_g = start_gather(0)
        cp_w = [None, None]
        for n in range(n_total):
            nxt = None
            if n + 1 < n_total:
                nb = (n + 1) % 2
                if cp_w[nb] is not None:
                    cp_w[nb].wait()
                    cp_w[nb] = None
                nxt = start_gather(n + 1)
            cp_g.wait()
            cp_w[n % 2] = start_write(n)
            cp_g = nxt
        for w in cp_w:
            if w is not None:
                w.wait()

    return gather_kernel(t0, t1, t2, cb0, cb1, cb2)


BB = 1024
EPS = 1e-5
SCALE = 1.0 / math.sqrt(DH)


def _tc_body(g_ref, wlo_ref, whi_ref, bqkv_ref, hm_ref, hmt_ref, wo_ref,
             bo_ref, gamma_ref, beta_ref, prev_ref, out_ref):
    gi = g_ref[...].reshape(S * BB, EP)
    elo = lax.bitcast_convert_type(gi << 16, jnp.float32)
    ehi = lax.bitcast_convert_type(gi & jnp.int32(-65536),
                                   jnp.float32)[:, :EH]
    e = jnp.concatenate([elo, ehi], axis=1)
    qkv = (jnp.dot(elo.astype(jnp.bfloat16), wlo_ref[...],
                   preferred_element_type=jnp.float32)
           + jnp.dot(ehi.astype(jnp.bfloat16), whi_ref[...],
                     preferred_element_type=jnp.float32)
           ).astype(jnp.bfloat16) + bqkv_ref[...]
    q = [qkv[i * BB:(i + 1) * BB, 0:E] for i in range(S)]
    k = [qkv[i * BB:(i + 1) * BB, E:2 * E] for i in range(S)]
    v = [qkv[i * BB:(i + 1) * BB, 2 * E:3 * E] for i in range(S)]

    hm = hm_ref[...]
    hmt = hmt_ref[...]
    s = [[jnp.dot(q[i] * k[j], hm, preferred_element_type=jnp.float32)
          for j in range(S)] for i in range(S)]

    acc_y = None
    for i in range(S):
        ex = [jnp.exp(s[i][j]) for j in range(S)]
        inv = 1.0 / (ex[0] + ex[1] + ex[2])
        acc = None
        for j in range(S):
            a = (ex[j] * inv).astype(jnp.bfloat16)
            aexp = jnp.dot(a, hmt,
                           preferred_element_type=jnp.float32
                           ).astype(jnp.bfloat16)
            av = (aexp * v[j]).astype(jnp.float32)
            acc = av if acc is None else acc + av
        x = jnp.dot(acc.astype(jnp.bfloat16), wo_ref[...],
                    preferred_element_type=jnp.float32) + bo_ref[...]             + e[i * BB:(i + 1) * BB]
        mu = jnp.mean(x, axis=-1, keepdims=True)
        xc = x - mu
        var = jnp.mean(xc * xc, axis=-1, keepdims=True)
        y = xc * lax.rsqrt(var + EPS) * gamma_ref[...] + beta_ref[...]
        acc_y = y if acc_y is None else acc_y + y
    out_ref[...] = acc_y * (1.0 / 3.0)


def _tc_compute(g, wlo16, whi16, bqkv, hm16, hmt16, wo16, bo2, gamma2, beta2,
                prev, block0):
    rows = g.shape[1]
    n_blocks = rows // BB
    const = lambda b: (0, 0)
    return pl.pallas_call(
        _tc_body,
        grid=(n_blocks,),
        in_specs=[
            pl.BlockSpec((S, BB, EP), lambda b: (0, b, 0)),
            pl.BlockSpec((EP, 3 * E), const),
            pl.BlockSpec((EH, 3 * E), const),
            pl.BlockSpec((1, 3 * E), const),
            pl.BlockSpec((E, H), const),
            pl.BlockSpec((H, E), const),
            pl.BlockSpec((E, E), const),
            pl.BlockSpec((1, E), const),
            pl.BlockSpec((1, E), const),
            pl.BlockSpec((1, E), const),
            pl.BlockSpec(memory_space=pltpu.MemorySpace.HBM),
        ],
        out_specs=pl.BlockSpec((BB, E), lambda b, _b0=block0: (_b0 + b, 0)),
        out_shape=jax.ShapeDtypeStruct((B, E), jnp.float32),
        input_output_aliases={10: 0},
    )(g, wlo16, whi16, bqkv, hm16, hmt16, wo16, bo2, gamma2, beta2, prev)


NSPLIT = 4
CHUNK = B // NSPLIT


def _pack(cb):
    cb16 = cb.astype(jnp.bfloat16)
    lo = cb16[:, :EP]
    hi = jnp.pad(cb16[:, EP:], ((0, 0), (0, EP - EH)))
    return lax.bitcast_convert_type(jnp.stack([lo, hi], axis=-1), jnp.int32)


def kernel(tokens, cb0, cb1, cb2, Wq, bq, Wk, bk, Wv, bv, Wo, bo, gamma, beta):
    t0 = tokens[:, 0]
    t1 = tokens[:, 1]
    t2 = tokens[:, 2]
    cb0p = _pack(cb0)
    cb1p = _pack(cb1)
    cb2p = _pack(cb2)

    wqkv16 = jnp.concatenate([Wq.T, Wk.T, Wv.T], axis=1).astype(jnp.bfloat16)
    wlo16 = wqkv16[:EP]
    whi16 = wqkv16[EP:]
    bqkv = jnp.concatenate([bq, bk, bv]).reshape(1, 3 * E).astype(jnp.bfloat16)
    head_of = jnp.arange(E, dtype=jnp.int32) // DH
    hm = (head_of[:, None] == jnp.arange(H, dtype=jnp.int32)[None, :])
    hm16 = (hm.astype(jnp.float32) * SCALE).astype(jnp.bfloat16)
    hmt16 = hm.T.astype(jnp.bfloat16)
    wo16 = Wo.T.astype(jnp.bfloat16)
    bo2 = bo.reshape(1, E)
    gamma2 = gamma.reshape(1, E)
    beta2 = beta.reshape(1, E)

    feat = jnp.zeros((B, E), jnp.float32)
    for c in range(NSPLIT):
        sl = slice(c * CHUNK, (c + 1) * CHUNK)
        g = _sc_gather(t0[sl], t1[sl], t2[sl], cb0p, cb1p, cb2p, CHUNK)
        feat = _tc_compute(g, wlo16, whi16, bqkv, hm16, hmt16, wo16,
                           bo2, gamma2, beta2, feat, c * (CHUNK // BB))
    return feat


# speedup vs baseline: 1.2703x; 1.0277x over previous
import functools
import math

import jax
import jax.numpy as jnp
from jax import lax
from jax.experimental import pallas as pl
from jax.experimental.pallas import tpu as pltpu, tpu_sc as plsc

B = 16384
E = 384
EP = 256
EH = E - EP
H = 4
DH = E // H
K = 1024
S = 3

NC = 2
NS = 16
NW = NC * NS


def _sc_gather(t0, t1, t2, cb0, cb1, cb2, rows):
    B_PER_W = rows // NW
    CH = min(128, B_PER_W)
    NCHUNK = B_PER_W // CH
    mesh = plsc.VectorSubcoreMesh(
        core_axis_name="c", subcore_axis_name="s",
        num_cores=NC, num_subcores=NS)

    @functools.partial(
        pl.kernel,
        out_type=jax.ShapeDtypeStruct((S, rows, EP), jnp.int32),
        mesh=mesh,
        scratch_types=[
            pltpu.VMEM((B_PER_W,), jnp.int32),
            pltpu.VMEM((B_PER_W,), jnp.int32),
            pltpu.VMEM((B_PER_W,), jnp.int32),
            pltpu.VMEM((CH, EP), jnp.int32),
            pltpu.VMEM((CH, EP), jnp.int32),
            pltpu.SemaphoreType.DMA,
            pltpu.SemaphoreType.DMA,
            pltpu.SemaphoreType.DMA,
            pltpu.SemaphoreType.DMA,
        ],
    )
    def gather_kernel(t0_hbm, t1_hbm, t2_hbm, cb0_hbm, cb1_hbm, cb2_hbm,
                      out_hbm, idx0_v, idx1_v, idx2_v, buf0, buf1,
                      sg0, sg1, sw0, sw1):
        wid = lax.axis_index("s") * NC + lax.axis_index("c")
        base = wid * B_PER_W
        tok_refs = (t0_hbm, t1_hbm, t2_hbm)
        cb_refs = (cb0_hbm, cb1_hbm, cb2_hbm)
        idx_refs = (idx0_v, idx1_v, idx2_v)
        for ti in range(S):
            pltpu.sync_copy(tok_refs[ti].at[pl.ds(base, B_PER_W)],
                            idx_refs[ti])

        bufs = (buf0, buf1)
        gsems = (sg0, sg1)
        wsems = (sw0, sw1)
        chunks = [(ti, c) for ti in range(S) for c in range(NCHUNK)]
        n_total = len(chunks)

        def start_gather(n):
            ti, c = chunks[n]
            return pltpu.async_copy(
                cb_refs[ti].at[idx_refs[ti].at[pl.ds(c * CH, CH)]],
                bufs[n % 2], gsems[n % 2])

        def start_write(n):
            ti, c = chunks[n]
            return pltpu.async_copy(
                bufs[n % 2],
                out_hbm.at[ti, pl.ds(base + c * CH, CH)],
                wsems[n % 2])

        cp_g = start_gather(0)
        cp_w = [None, None]
        for n in range(n_total):
            nxt = None
            if n + 1 < n_total:
                nb = (n + 1) % 2
                if cp_w[nb] is not None:
                    cp_w[nb].wait()
                    cp_w[nb] = None
                nxt = start_gather(n + 1)
            cp_g.wait()
            cp_w[n % 2] = start_write(n)
            cp_g = nxt
        for w in cp_w:
            if w is not None:
                w.wait()

    return gather_kernel(t0, t1, t2, cb0, cb1, cb2)


BB = 1024
EPS = 1e-5
SCALE = 1.0 / math.sqrt(DH)


def _tc_body(g_ref, wlo_ref, whi_ref, bqkv_ref, hm_ref, hmt_ref, wo_ref,
             bo_ref, gamma_ref, beta_ref, prev_ref, out_ref):
    gi = g_ref[...].reshape(S * BB, EP)
    elo = lax.bitcast_convert_type(gi << 16, jnp.float32)
    ehi = lax.bitcast_convert_type(gi & jnp.int32(-65536),
                                   jnp.float32)[:, :EH]
    e = jnp.concatenate([elo, ehi], axis=1)
    qkv = (jnp.dot(elo.astype(jnp.bfloat16), wlo_ref[...],
                   preferred_element_type=jnp.float32)
           + jnp.dot(ehi.astype(jnp.bfloat16), whi_ref[...],
                     preferred_element_type=jnp.float32)
           ).astype(jnp.bfloat16) + bqkv_ref[...]
    q = [qkv[i * BB:(i + 1) * BB, 0:E] for i in range(S)]
    k = [qkv[i * BB:(i + 1) * BB, E:2 * E] for i in range(S)]
    v = [qkv[i * BB:(i + 1) * BB, 2 * E:3 * E] for i in range(S)]

    hm = hm_ref[...]
    hmt = hmt_ref[...]
    s = [[jnp.dot(q[i] * k[j], hm, preferred_element_type=jnp.float32)
          for j in range(S)] for i in range(S)]

    acc_y = None
    for i in range(S):
        ex = [jnp.exp(s[i][j]) for j in range(S)]
        inv = 1.0 / (ex[0] + ex[1] + ex[2])
        acc = None
        for j in range(S):
            a = (ex[j] * inv).astype(jnp.bfloat16)
            aexp = jnp.dot(a, hmt,
                           preferred_element_type=jnp.float32
                           ).astype(jnp.bfloat16)
            av = aexp * v[j]
            acc = av if acc is None else acc + av
        x = jnp.dot(acc, wo_ref[...],
                    preferred_element_type=jnp.float32) + bo_ref[...]             + e[i * BB:(i + 1) * BB]
        mu = jnp.mean(x, axis=-1, keepdims=True)
        xc = x - mu
        var = jnp.mean(xc * xc, axis=-1, keepdims=True)
        y = xc * lax.rsqrt(var + EPS) * gamma_ref[...] + beta_ref[...]
        acc_y = y if acc_y is None else acc_y + y
    out_ref[...] = acc_y * (1.0 / 3.0)


def _tc_compute(g, wlo16, whi16, bqkv, hm16, hmt16, wo16, bo2, gamma2, beta2,
                prev, block0):
    rows = g.shape[1]
    n_blocks = rows // BB
    const = lambda b: (0, 0)
    return pl.pallas_call(
        _tc_body,
        grid=(n_blocks,),
        in_specs=[
            pl.BlockSpec((S, BB, EP), lambda b: (0, b, 0)),
            pl.BlockSpec((EP, 3 * E), const),
            pl.BlockSpec((EH, 3 * E), const),
            pl.BlockSpec((1, 3 * E), const),
            pl.BlockSpec((E, H), const),
            pl.BlockSpec((H, E), const),
            pl.BlockSpec((E, E), const),
            pl.BlockSpec((1, E), const),
            pl.BlockSpec((1, E), const),
            pl.BlockSpec((1, E), const),
            pl.BlockSpec(memory_space=pltpu.MemorySpace.HBM),
        ],
        out_specs=pl.BlockSpec((BB, E), lambda b, _b0=block0: (_b0 + b, 0)),
        out_shape=jax.ShapeDtypeStruct((B, E), jnp.float32),
        input_output_aliases={10: 0},
    )(g, wlo16, whi16, bqkv, hm16, hmt16, wo16, bo2, gamma2, beta2, prev)


NSPLIT = 4
CHUNK = B // NSPLIT


def _pack(cb):
    cb16 = cb.astype(jnp.bfloat16)
    lo = cb16[:, :EP]
    hi = jnp.pad(cb16[:, EP:], ((0, 0), (0, EP - EH)))
    return lax.bitcast_convert_type(jnp.stack([lo, hi], axis=-1), jnp.int32)


def kernel(tokens, cb0, cb1, cb2, Wq, bq, Wk, bk, Wv, bv, Wo, bo, gamma, beta):
    t0 = tokens[:, 0]
    t1 = tokens[:, 1]
    t2 = tokens[:, 2]
    cb0p = _pack(cb0)
    cb1p = _pack(cb1)
    cb2p = _pack(cb2)

    wqkv16 = jnp.concatenate([Wq.T, Wk.T, Wv.T], axis=1).astype(jnp.bfloat16)
    wlo16 = wqkv16[:EP]
    whi16 = wqkv16[EP:]
    bqkv = jnp.concatenate([bq, bk, bv]).reshape(1, 3 * E).astype(jnp.bfloat16)
    head_of = jnp.arange(E, dtype=jnp.int32) // DH
    hm = (head_of[:, None] == jnp.arange(H, dtype=jnp.int32)[None, :])
    hm16 = (hm.astype(jnp.float32) * SCALE).astype(jnp.bfloat16)
    hmt16 = hm.T.astype(jnp.bfloat16)
    wo16 = Wo.T.astype(jnp.bfloat16)
    bo2 = bo.reshape(1, E)
    gamma2 = gamma.reshape(1, E)
    beta2 = beta.reshape(1, E)

    feat = jnp.zeros((B, E), jnp.float32)
    for c in range(NSPLIT):
        sl = slice(c * CHUNK, (c + 1) * CHUNK)
        g = _sc_gather(t0[sl], t1[sl], t2[sl], cb0p, cb1p, cb2p, CHUNK)
        feat = _tc_compute(g, wlo16, whi16, bqkv, hm16, hmt16, wo16,
                           bo2, gamma2, beta2, feat, c * (CHUNK // BB))
    return feat


# speedup vs baseline: 1.3172x; 1.0369x over previous
import functools
import math

import jax
import jax.numpy as jnp
from jax import lax
from jax.experimental import pallas as pl
from jax.experimental.pallas import tpu as pltpu, tpu_sc as plsc

B = 16384
E = 384
EP = 256
EH = E - EP
H = 4
DH = E // H
K = 1024
S = 3

NC = 2
NS = 16
NW = NC * NS


def _sc_gather(t0, t1, t2, cb0, cb1, cb2, rows):
    B_PER_W = rows // NW
    CH = min(128, B_PER_W)
    NCHUNK = B_PER_W // CH
    mesh = plsc.VectorSubcoreMesh(
        core_axis_name="c", subcore_axis_name="s",
        num_cores=NC, num_subcores=NS)

    @functools.partial(
        pl.kernel,
        out_type=jax.ShapeDtypeStruct((S, rows, EP), jnp.int32),
        mesh=mesh,
        scratch_types=[
            pltpu.VMEM((B_PER_W,), jnp.int32),
            pltpu.VMEM((B_PER_W,), jnp.int32),
            pltpu.VMEM((B_PER_W,), jnp.int32),
            pltpu.VMEM((CH, EP), jnp.int32),
            pltpu.VMEM((CH, EP), jnp.int32),
            pltpu.SemaphoreType.DMA,
            pltpu.SemaphoreType.DMA,
            pltpu.SemaphoreType.DMA,
            pltpu.SemaphoreType.DMA,
        ],
    )
    def gather_kernel(t0_hbm, t1_hbm, t2_hbm, cb0_hbm, cb1_hbm, cb2_hbm,
                      out_hbm, idx0_v, idx1_v, idx2_v, buf0, buf1,
                      sg0, sg1, sw0, sw1):
        wid = lax.axis_index("s") * NC + lax.axis_index("c")
        base = wid * B_PER_W
        tok_refs = (t0_hbm, t1_hbm, t2_hbm)
        cb_refs = (cb0_hbm, cb1_hbm, cb2_hbm)
        idx_refs = (idx0_v, idx1_v, idx2_v)
        for ti in range(S):
            pltpu.sync_copy(tok_refs[ti].at[pl.ds(base, B_PER_W)],
                            idx_refs[ti])

        bufs = (buf0, buf1)
        gsems = (sg0, sg1)
        wsems = (sw0, sw1)
        chunks = [(ti, c) for ti in range(S) for c in range(NCHUNK)]
        n_total = len(chunks)

        def start_gather(n):
            ti, c = chunks[n]
            return pltpu.async_copy(
                cb_refs[ti].at[idx_refs[ti].at[pl.ds(c * CH, CH)]],
                bufs[n % 2], gsems[n % 2])

        def start_write(n):
            ti, c = chunks[n]
            return pltpu.async_copy(
                bufs[n % 2],
                out_hbm.at[ti, pl.ds(base + c * CH, CH)],
                wsems[n % 2])

        cp_g = start_gather(0)
        cp_w = [None, None]
        for n in range(n_total):
            nxt = None
            if n + 1 < n_total:
                nb = (n + 1) % 2
                if cp_w[nb] is not None:
                    cp_w[nb].wait()
                    cp_w[nb] = None
                nxt = start_gather(n + 1)
            cp_g.wait()
            cp_w[n % 2] = start_write(n)
            cp_g = nxt
        for w in cp_w:
            if w is not None:
                w.wait()

    return gather_kernel(t0, t1, t2, cb0, cb1, cb2)


BB = 1024
EPS = 1e-5
SCALE = 1.0 / math.sqrt(DH)


def _tc_body(g_ref, wlo_ref, whi_ref, bqkv_ref, hm_ref, hmt_ref, wo_ref,
             bo_ref, gamma_ref, beta_ref, prev_ref, out_ref):
    _tc_body_first(g_ref, wlo_ref, whi_ref, bqkv_ref, hm_ref, hmt_ref,
                   wo_ref, bo_ref, gamma_ref, beta_ref, out_ref)


def _tc_body_first(g_ref, wlo_ref, whi_ref, bqkv_ref, hm_ref, hmt_ref, wo_ref,
                   bo_ref, gamma_ref, beta_ref, out_ref):
    gi = g_ref[...].reshape(S * BB, EP)
    elo = lax.bitcast_convert_type(gi << 16, jnp.float32)
    ehi = lax.bitcast_convert_type(gi & jnp.int32(-65536),
                                   jnp.float32)[:, :EH]
    e = jnp.concatenate([elo, ehi], axis=1)
    qkv = (jnp.dot(elo.astype(jnp.bfloat16), wlo_ref[...],
                   preferred_element_type=jnp.float32)
           + jnp.dot(ehi.astype(jnp.bfloat16), whi_ref[...],
                     preferred_element_type=jnp.float32)
           ).astype(jnp.bfloat16) + bqkv_ref[...]
    q = [qkv[i * BB:(i + 1) * BB, 0:E] for i in range(S)]
    k = [qkv[i * BB:(i + 1) * BB, E:2 * E] for i in range(S)]
    v = [qkv[i * BB:(i + 1) * BB, 2 * E:3 * E] for i in range(S)]

    hm = hm_ref[...]
    hmt = hmt_ref[...]
    s = [[jnp.dot(q[i] * k[j], hm, preferred_element_type=jnp.float32)
          for j in range(S)] for i in range(S)]

    acc_y = None
    for i in range(S):
        ex = [jnp.exp(s[i][j]) for j in range(S)]
        inv = 1.0 / (ex[0] + ex[1] + ex[2])
        acc = None
        for j in range(S):
            a = (ex[j] * inv).astype(jnp.bfloat16)
            aexp = jnp.dot(a, hmt,
                           preferred_element_type=jnp.float32
                           ).astype(jnp.bfloat16)
            av = aexp * v[j]
            acc = av if acc is None else acc + av
        x = jnp.dot(acc, wo_ref[...],
                    preferred_element_type=jnp.float32) + bo_ref[...]             + e[i * BB:(i + 1) * BB]
        mu = jnp.mean(x, axis=-1, keepdims=True)
        xc = x - mu
        var = jnp.mean(xc * xc, axis=-1, keepdims=True)
        y = xc * lax.rsqrt(var + EPS) * gamma_ref[...] + beta_ref[...]
        acc_y = y if acc_y is None else acc_y + y
    out_ref[...] = acc_y * (1.0 / 3.0)


def _tc_compute(g, wlo16, whi16, bqkv, hm16, hmt16, wo16, bo2, gamma2, beta2,
                prev, block0):
    rows = g.shape[1]
    n_blocks = rows // BB
    const = lambda b: (0, 0)
    in_specs = [
        pl.BlockSpec((S, BB, EP), lambda b: (0, b, 0)),
        pl.BlockSpec((EP, 3 * E), const),
        pl.BlockSpec((EH, 3 * E), const),
        pl.BlockSpec((1, 3 * E), const),
        pl.BlockSpec((E, H), const),
        pl.BlockSpec((H, E), const),
        pl.BlockSpec((E, E), const),
        pl.BlockSpec((1, E), const),
        pl.BlockSpec((1, E), const),
        pl.BlockSpec((1, E), const),
    ]
    args = [g, wlo16, whi16, bqkv, hm16, hmt16, wo16, bo2, gamma2, beta2]
    aliases = {}
    body = _tc_body_first
    if prev is not None:
        in_specs.append(pl.BlockSpec(memory_space=pltpu.MemorySpace.HBM))
        args.append(prev)
        aliases = {10: 0}
        body = _tc_body
    return pl.pallas_call(
        body,
        grid=(n_blocks,),
        in_specs=in_specs,
        out_specs=pl.BlockSpec((BB, E), lambda b, _b0=block0: (_b0 + b, 0)),
        out_shape=jax.ShapeDtypeStruct((B, E), jnp.float32),
        input_output_aliases=aliases,
    )(*args)


NSPLIT = 4
CHUNK = B // NSPLIT


def _pack(cb):
    cb16 = cb.astype(jnp.bfloat16)
    lo = cb16[:, :EP]
    hi = jnp.pad(cb16[:, EP:], ((0, 0), (0, EP - EH)))
    return lax.bitcast_convert_type(jnp.stack([lo, hi], axis=-1), jnp.int32)


def kernel(tokens, cb0, cb1, cb2, Wq, bq, Wk, bk, Wv, bv, Wo, bo, gamma, beta):
    t0 = tokens[:, 0]
    t1 = tokens[:, 1]
    t2 = tokens[:, 2]
    cb0p = _pack(cb0)
    cb1p = _pack(cb1)
    cb2p = _pack(cb2)

    wqkv16 = jnp.concatenate([Wq.T, Wk.T, Wv.T], axis=1).astype(jnp.bfloat16)
    wlo16 = wqkv16[:EP]
    whi16 = wqkv16[EP:]
    bqkv = jnp.concatenate([bq, bk, bv]).reshape(1, 3 * E).astype(jnp.bfloat16)
    head_of = jnp.arange(E, dtype=jnp.int32) // DH
    hm = (head_of[:, None] == jnp.arange(H, dtype=jnp.int32)[None, :])
    hm16 = (hm.astype(jnp.float32) * SCALE).astype(jnp.bfloat16)
    hmt16 = hm.T.astype(jnp.bfloat16)
    wo16 = Wo.T.astype(jnp.bfloat16)
    bo2 = bo.reshape(1, E)
    gamma2 = gamma.reshape(1, E)
    beta2 = beta.reshape(1, E)

    feat = None
    for c in range(NSPLIT):
        sl = slice(c * CHUNK, (c + 1) * CHUNK)
        g = _sc_gather(t0[sl], t1[sl], t2[sl], cb0p, cb1p, cb2p, CHUNK)
        feat = _tc_compute(g, wlo16, whi16, bqkv, hm16, hmt16, wo16,
                           bo2, gamma2, beta2, feat, c * (CHUNK // BB))
    return feat


# speedup vs baseline: 1.3330x; 1.0120x over previous
import functools
import math

import jax
import jax.numpy as jnp
from jax import lax
from jax.experimental import pallas as pl
from jax.experimental.pallas import tpu as pltpu, tpu_sc as plsc

B = 16384
E = 384
EP = 256
EH = E - EP
H = 4
DH = E // H
K = 1024
S = 3

NC = 2
NS = 16
NW = NC * NS


def _sc_gather(t0, t1, t2, cb0, cb1, cb2, rows):
    B_PER_W = rows // NW
    CH = min(128, B_PER_W)
    NCHUNK = B_PER_W // CH
    mesh = plsc.VectorSubcoreMesh(
        core_axis_name="c", subcore_axis_name="s",
        num_cores=NC, num_subcores=NS)

    @functools.partial(
        pl.kernel,
        out_type=jax.ShapeDtypeStruct((S, rows, EP), jnp.int32),
        mesh=mesh,
        scratch_types=[
            pltpu.VMEM((B_PER_W,), jnp.int32),
            pltpu.VMEM((B_PER_W,), jnp.int32),
            pltpu.VMEM((B_PER_W,), jnp.int32),
            pltpu.VMEM((CH, EP), jnp.int32),
            pltpu.VMEM((CH, EP), jnp.int32),
            pltpu.SemaphoreType.DMA,
            pltpu.SemaphoreType.DMA,
            pltpu.SemaphoreType.DMA,
            pltpu.SemaphoreType.DMA,
        ],
    )
    def gather_kernel(t0_hbm, t1_hbm, t2_hbm, cb0_hbm, cb1_hbm, cb2_hbm,
                      out_hbm, idx0_v, idx1_v, idx2_v, buf0, buf1,
                      sg0, sg1, sw0, sw1):
        wid = lax.axis_index("s") * NC + lax.axis_index("c")
        base = wid * B_PER_W
        tok_refs = (t0_hbm, t1_hbm, t2_hbm)
        cb_refs = (cb0_hbm, cb1_hbm, cb2_hbm)
        idx_refs = (idx0_v, idx1_v, idx2_v)
        for ti in range(S):
            pltpu.sync_copy(tok_refs[ti].at[pl.ds(base, B_PER_W)],
                            idx_refs[ti])

        bufs = (buf0, buf1)
        gsems = (sg0, sg1)
        wsems = (sw0, sw1)
        chunks = [(ti, c) for ti in range(S) for c in range(NCHUNK)]
        n_total = len(chunks)

        def start_gather(n):
            ti, c = chunks[n]
            return pltpu.async_copy(
                cb_refs[ti].at[idx_refs[ti].at[pl.ds(c * CH, CH)]],
                bufs[n % 2], gsems[n % 2])

        def start_write(n):
            ti, c = chunks[n]
            return pltpu.async_copy(
                bufs[n % 2],
                out_hbm.at[ti, pl.ds(base + c * CH, CH)],
                wsems[n % 2])

        cp_g = start_gather(0)
        cp_w = [None, None]
        for n in range(n_total):
            nxt = None
            if n + 1 < n_total:
                nb = (n + 1) % 2
                if cp_w[nb] is not None:
                    cp_w[nb].wait()
                    cp_w[nb] = None
                nxt = start_gather(n + 1)
            cp_g.wait()
            cp_w[n % 2] = start_write(n)
            cp_g = nxt
        for w in cp_w:
            if w is not None:
                w.wait()

    return gather_kernel(t0, t1, t2, cb0, cb1, cb2)


BB = 1024
EPS = 1e-5
SCALE = 1.0 / math.sqrt(DH)


def _tc_body(g_ref, wlo_ref, whi_ref, bqkv_ref, hm_ref, hmt_ref, wo_ref,
             bo_ref, gamma_ref, beta_ref, prev_ref, out_ref):
    _tc_body_first(g_ref, wlo_ref, whi_ref, bqkv_ref, hm_ref, hmt_ref,
                   wo_ref, bo_ref, gamma_ref, beta_ref, out_ref)


def _tc_body_first(g_ref, wlo_ref, whi_ref, bqkv_ref, hm_ref, hmt_ref, wo_ref,
                   bo_ref, gamma_ref, beta_ref, out_ref):
    gi = g_ref[...].reshape(S * BB, EP)
    elo = lax.bitcast_convert_type(gi << 16, jnp.float32)
    ehi = lax.bitcast_convert_type(gi & jnp.int32(-65536),
                                   jnp.float32)[:, :EH]
    e = jnp.concatenate([elo, ehi], axis=1)
    qkv = (jnp.dot(elo.astype(jnp.bfloat16), wlo_ref[...],
                   preferred_element_type=jnp.float32)
           + jnp.dot(ehi.astype(jnp.bfloat16), whi_ref[...],
                     preferred_element_type=jnp.float32)
           ).astype(jnp.bfloat16) + bqkv_ref[...]
    q = [qkv[i * BB:(i + 1) * BB, 0:E] for i in range(S)]
    k = [qkv[i * BB:(i + 1) * BB, E:2 * E] for i in range(S)]
    v = [qkv[i * BB:(i + 1) * BB, 2 * E:3 * E] for i in range(S)]

    hm = hm_ref[...]
    hmt = hmt_ref[...]
    s = [[jnp.dot(q[i] * k[j], hm, preferred_element_type=jnp.float32)
          for j in range(S)] for i in range(S)]

    acc_y = None
    for i in range(S):
        ex = [jnp.exp(s[i][j]) for j in range(S)]
        inv = 1.0 / (ex[0] + ex[1] + ex[2])
        acc = None
        for j in range(S):
            a = (ex[j] * inv).astype(jnp.bfloat16)
            aexp = jnp.dot(a, hmt,
                           preferred_element_type=jnp.float32
                           ).astype(jnp.bfloat16)
            av = aexp * v[j]
            acc = av if acc is None else acc + av
        x = jnp.dot(acc, wo_ref[...],
                    preferred_element_type=jnp.float32) + bo_ref[...]             + e[i * BB:(i + 1) * BB]
        mu = jnp.mean(x, axis=-1, keepdims=True)
        xc = x - mu
        var = jnp.mean(xc * xc, axis=-1, keepdims=True)
        y = xc * lax.rsqrt(var + EPS) * gamma_ref[...] + beta_ref[...]
        acc_y = y if acc_y is None else acc_y + y
    out_ref[...] = acc_y * (1.0 / 3.0)


def _tc_compute(g, wlo16, whi16, bqkv, hm16, hmt16, wo16, bo2, gamma2, beta2,
                prev, block0):
    rows = g.shape[1]
    n_blocks = rows // BB
    const = lambda b: (0, 0)
    in_specs = [
        pl.BlockSpec((S, BB, EP), lambda b: (0, b, 0)),
        pl.BlockSpec((EP, 3 * E), const),
        pl.BlockSpec((EH, 3 * E), const),
        pl.BlockSpec((1, 3 * E), const),
        pl.BlockSpec((E, H), const),
        pl.BlockSpec((H, E), const),
        pl.BlockSpec((E, E), const),
        pl.BlockSpec((1, E), const),
        pl.BlockSpec((1, E), const),
        pl.BlockSpec((1, E), const),
    ]
    args = [g, wlo16, whi16, bqkv, hm16, hmt16, wo16, bo2, gamma2, beta2]
    aliases = {}
    body = _tc_body_first
    if prev is not None:
        in_specs.append(pl.BlockSpec(memory_space=pltpu.MemorySpace.HBM))
        args.append(prev)
        aliases = {10: 0}
        body = _tc_body
    return pl.pallas_call(
        body,
        grid=(n_blocks,),
        in_specs=in_specs,
        out_specs=pl.BlockSpec((BB, E), lambda b, _b0=block0: (_b0 + b, 0)),
        out_shape=jax.ShapeDtypeStruct((B, E), jnp.float32),
        input_output_aliases=aliases,
    )(*args)


CHUNKS = (2048, 2048, 4096, 8192)


def _pack(cb):
    cb16 = cb.astype(jnp.bfloat16)
    lo = cb16[:, :EP]
    hi = jnp.pad(cb16[:, EP:], ((0, 0), (0, EP - EH)))
    return lax.bitcast_convert_type(jnp.stack([lo, hi], axis=-1), jnp.int32)


def kernel(tokens, cb0, cb1, cb2, Wq, bq, Wk, bk, Wv, bv, Wo, bo, gamma, beta):
    t0 = tokens[:, 0]
    t1 = tokens[:, 1]
    t2 = tokens[:, 2]
    cb0p = _pack(cb0)
    cb1p = _pack(cb1)
    cb2p = _pack(cb2)

    wqkv16 = jnp.concatenate([Wq.T, Wk.T, Wv.T], axis=1).astype(jnp.bfloat16)
    wlo16 = wqkv16[:EP]
    whi16 = wqkv16[EP:]
    bqkv = jnp.concatenate([bq, bk, bv]).reshape(1, 3 * E).astype(jnp.bfloat16)
    head_of = jnp.arange(E, dtype=jnp.int32) // DH
    hm = (head_of[:, None] == jnp.arange(H, dtype=jnp.int32)[None, :])
    hm16 = (hm.astype(jnp.float32) * SCALE).astype(jnp.bfloat16)
    hmt16 = hm.T.astype(jnp.bfloat16)
    wo16 = Wo.T.astype(jnp.bfloat16)
    bo2 = bo.reshape(1, E)
    gamma2 = gamma.reshape(1, E)
    beta2 = beta.reshape(1, E)

    feat = None
    row0 = 0
    for rows in CHUNKS:
        sl = slice(row0, row0 + rows)
        g = _sc_gather(t0[sl], t1[sl], t2[sl], cb0p, cb1p, cb2p, rows)
        feat = _tc_compute(g, wlo16, whi16, bqkv, hm16, hmt16, wo16,
                           bo2, gamma2, beta2, feat, row0 // BB)
        row0 += rows
    return feat


# speedup vs baseline: 1.3342x; 1.0009x over previous
import functools
import math

import jax
import jax.numpy as jnp
from jax import lax
from jax.experimental import pallas as pl
from jax.experimental.pallas import tpu as pltpu, tpu_sc as plsc

B = 16384
E = 384
EP = 256
EH = E - EP
H = 4
DH = E // H
K = 1024
S = 3

NC = 2
NS = 16
NW = NC * NS


def _sc_gather(t0, t1, t2, cb0, cb1, cb2, rows):
    B_PER_W = rows // NW
    CH = min(128, B_PER_W)
    NCHUNK = B_PER_W // CH
    mesh = plsc.VectorSubcoreMesh(
        core_axis_name="c", subcore_axis_name="s",
        num_cores=NC, num_subcores=NS)

    @functools.partial(
        pl.kernel,
        out_type=jax.ShapeDtypeStruct((S, rows, EP), jnp.int32),
        mesh=mesh,
        scratch_types=[
            pltpu.VMEM((B_PER_W,), jnp.int32),
            pltpu.VMEM((B_PER_W,), jnp.int32),
            pltpu.VMEM((B_PER_W,), jnp.int32),
            pltpu.VMEM((CH, EP), jnp.int32),
            pltpu.VMEM((CH, EP), jnp.int32),
            pltpu.SemaphoreType.DMA,
            pltpu.SemaphoreType.DMA,
            pltpu.SemaphoreType.DMA,
            pltpu.SemaphoreType.DMA,
        ],
    )
    def gather_kernel(t0_hbm, t1_hbm, t2_hbm, cb0_hbm, cb1_hbm, cb2_hbm,
                      out_hbm, idx0_v, idx1_v, idx2_v, buf0, buf1,
                      sg0, sg1, sw0, sw1):
        wid = lax.axis_index("s") * NC + lax.axis_index("c")
        base = wid * B_PER_W
        tok_refs = (t0_hbm, t1_hbm, t2_hbm)
        cb_refs = (cb0_hbm, cb1_hbm, cb2_hbm)
        idx_refs = (idx0_v, idx1_v, idx2_v)
        for ti in range(S):
            pltpu.sync_copy(tok_refs[ti].at[pl.ds(base, B_PER_W)],
                            idx_refs[ti])

        bufs = (buf0, buf1)
        gsems = (sg0, sg1)
        wsems = (sw0, sw1)
        chunks = [(ti, c) for ti in range(S) for c in range(NCHUNK)]
        n_total = len(chunks)

        def start_gather(n):
            ti, c = chunks[n]
            return pltpu.async_copy(
                cb_refs[ti].at[idx_refs[ti].at[pl.ds(c * CH, CH)]],
                bufs[n % 2], gsems[n % 2])

        def start_write(n):
            ti, c = chunks[n]
            return pltpu.async_copy(
                bufs[n % 2],
                out_hbm.at[ti, pl.ds(base + c * CH, CH)],
                wsems[n % 2])

        cp_g = start_gather(0)
        cp_w = [None, None]
        for n in range(n_total):
            nxt = None
            if n + 1 < n_total:
                nb = (n + 1) % 2
                if cp_w[nb] is not None:
                    cp_w[nb].wait()
                    cp_w[nb] = None
                nxt = start_gather(n + 1)
            cp_g.wait()
            cp_w[n % 2] = start_write(n)
            cp_g = nxt
        for w in cp_w:
            if w is not None:
                w.wait()

    return gather_kernel(t0, t1, t2, cb0, cb1, cb2)


BB = 1024
EPS = 1e-5
SCALE = 1.0 / math.sqrt(DH)


def _tc_body(g_ref, wlo_ref, whi_ref, bqkv_ref, hm_ref, hmt_ref, wo_ref,
             bo_ref, gamma_ref, beta_ref, prev_ref, out_ref):
    _tc_body_first(g_ref, wlo_ref, whi_ref, bqkv_ref, hm_ref, hmt_ref,
                   wo_ref, bo_ref, gamma_ref, beta_ref, out_ref)


def _tc_body_first(g_ref, wlo_ref, whi_ref, bqkv_ref, hm_ref, hmt_ref, wo_ref,
                   bo_ref, gamma_ref, beta_ref, out_ref):
    gi = g_ref[...].reshape(S * BB, EP)
    elo = lax.bitcast_convert_type(gi << 16, jnp.float32)
    ehi = lax.bitcast_convert_type(gi & jnp.int32(-65536),
                                   jnp.float32)[:, :EH]
    e = jnp.concatenate([elo, ehi], axis=1)
    qkv = (jnp.dot(elo.astype(jnp.bfloat16), wlo_ref[...],
                   preferred_element_type=jnp.float32)
           + jnp.dot(ehi.astype(jnp.bfloat16), whi_ref[...],
                     preferred_element_type=jnp.float32)
           ).astype(jnp.bfloat16) + bqkv_ref[...]
    q = [qkv[i * BB:(i + 1) * BB, 0:E] for i in range(S)]
    k = [qkv[i * BB:(i + 1) * BB, E:2 * E] for i in range(S)]
    v = [qkv[i * BB:(i + 1) * BB, 2 * E:3 * E] for i in range(S)]

    hm = hm_ref[...]
    hmt = hmt_ref[...]
    s = [[jnp.dot(q[i] * k[j], hm, preferred_element_type=jnp.float32)
          for j in range(S)] for i in range(S)]

    acc_y = None
    for i in range(S):
        ex = [jnp.exp(s[i][j]) for j in range(S)]
        inv = 1.0 / (ex[0] + ex[1] + ex[2])
        acc = None
        for j in range(S):
            a = (ex[j] * inv).astype(jnp.bfloat16)
            aexp = jnp.dot(a, hmt,
                           preferred_element_type=jnp.float32
                           ).astype(jnp.bfloat16)
            av = aexp * v[j]
            acc = av if acc is None else acc + av
        x = jnp.dot(acc, wo_ref[...],
                    preferred_element_type=jnp.float32) + bo_ref[...]             + e[i * BB:(i + 1) * BB]
        mu = jnp.mean(x, axis=-1, keepdims=True)
        xc = x - mu
        var = jnp.mean(xc * xc, axis=-1, keepdims=True)
        y = xc * lax.rsqrt(var + EPS) * gamma_ref[...] + beta_ref[...]
        acc_y = y if acc_y is None else acc_y + y
    out_ref[...] = acc_y * (1.0 / 3.0)


def _tc_compute(g, wlo16, whi16, bqkv, hm16, hmt16, wo16, bo2, gamma2, beta2,
                prev, block0):
    rows = g.shape[1]
    n_blocks = rows // BB
    const = lambda b: (0, 0)
    in_specs = [
        pl.BlockSpec((S, BB, EP), lambda b: (0, b, 0)),
        pl.BlockSpec((EP, 3 * E), const),
        pl.BlockSpec((EH, 3 * E), const),
        pl.BlockSpec((1, 3 * E), const),
        pl.BlockSpec((E, H), const),
        pl.BlockSpec((H, E), const),
        pl.BlockSpec((E, E), const),
        pl.BlockSpec((1, E), const),
        pl.BlockSpec((1, E), const),
        pl.BlockSpec((1, E), const),
    ]
    args = [g, wlo16, whi16, bqkv, hm16, hmt16, wo16, bo2, gamma2, beta2]
    aliases = {}
    body = _tc_body_first
    if prev is not None:
        in_specs.append(pl.BlockSpec(memory_space=pltpu.MemorySpace.HBM))
        args.append(prev)
        aliases = {10: 0}
        body = _tc_body
    return pl.pallas_call(
        body,
        grid=(n_blocks,),
        in_specs=in_specs,
        out_specs=pl.BlockSpec((BB, E), lambda b, _b0=block0: (_b0 + b, 0)),
        out_shape=jax.ShapeDtypeStruct((B, E), jnp.float32),
        input_output_aliases=aliases,
        compiler_params=pltpu.CompilerParams(
            dimension_semantics=("parallel",)),
    )(*args)


CHUNKS = (2048, 2048, 4096, 8192)


def _pack(cb):
    cb16 = cb.astype(jnp.bfloat16)
    lo = cb16[:, :EP]
    hi = jnp.pad(cb16[:, EP:], ((0, 0), (0, EP - EH)))
    return lax.bitcast_convert_type(jnp.stack([lo, hi], axis=-1), jnp.int32)


def kernel(tokens, cb0, cb1, cb2, Wq, bq, Wk, bk, Wv, bv, Wo, bo, gamma, beta):
    t0 = tokens[:, 0]
    t1 = tokens[:, 1]
    t2 = tokens[:, 2]
    cb0p = _pack(cb0)
    cb1p = _pack(cb1)
    cb2p = _pack(cb2)

    wqkv16 = jnp.concatenate([Wq.T, Wk.T, Wv.T], axis=1).astype(jnp.bfloat16)
    wlo16 = wqkv16[:EP]
    whi16 = wqkv16[EP:]
    bqkv = jnp.concatenate([bq, bk, bv]).reshape(1, 3 * E).astype(jnp.bfloat16)
    head_of = jnp.arange(E, dtype=jnp.int32) // DH
    hm = (head_of[:, None] == jnp.arange(H, dtype=jnp.int32)[None, :])
    hm16 = (hm.astype(jnp.float32) * SCALE).astype(jnp.bfloat16)
    hmt16 = hm.T.astype(jnp.bfloat16)
    wo16 = Wo.T.astype(jnp.bfloat16)
    bo2 = bo.reshape(1, E)
    gamma2 = gamma.reshape(1, E)
    beta2 = beta.reshape(1, E)

    feat = None
    row0 = 0
    for rows in CHUNKS:
        sl = slice(row0, row0 + rows)
        g = _sc_gather(t0[sl], t1[sl], t2[sl], cb0p, cb1p, cb2p, rows)
        feat = _tc_compute(g, wlo16, whi16, bqkv, hm16, hmt16, wo16,
                           bo2, gamma2, beta2, feat, row0 // BB)
        row0 += rows
    return feat
